# baseline (speedup 1.0000x reference)
import functools

import jax
import jax.numpy as jnp
from jax.experimental import pallas as pl
from jax.experimental.pallas import tpu as pltpu

_ROWS = 64
_NCOLS = 1_000_000
_W = 8192
_GRID = (_NCOLS + _W - 1) // _W


def _threefry_bits(flat):
    ks0 = jnp.uint32(0)
    ks1 = jnp.uint32(1234)
    ks2 = jnp.uint32(0 ^ 1234 ^ 0x1BD11BDA)
    ks = (ks0, ks1, ks2)
    rot = ((13, 15, 26, 6), (17, 29, 16, 24))
    x0 = jnp.full_like(flat, ks0)
    x1 = flat + ks1
    for i in range(5):
        for d in rot[i % 2]:
            x0 = x0 + x1
            x1 = (x1 << d) | (x1 >> (32 - d))
            x1 = x1 ^ x0
        x0 = x0 + ks[(i + 1) % 3]
        x1 = x1 + ks[(i + 2) % 3] + jnp.uint32(i + 1)
    return x0 ^ x1


def _body(logits_ref, temp_ref, idx_ref, bestv_ref):
    j = pl.program_id(0)
    s = logits_ref[...] / temp_ref[...]

    col = jax.lax.broadcasted_iota(jnp.int32, (_ROWS, _W), 1) + j * _W
    row = jax.lax.broadcasted_iota(jnp.int32, (_ROWS, _W), 0)
    flat = (row * _NCOLS + col).astype(jnp.uint32)

    bits = _threefry_bits(flat)
    fb = (bits >> 9) | jnp.uint32(0x3F800000)
    u = jax.lax.bitcast_convert_type(fb, jnp.float32) - 1.0
    noise = jnp.maximum(-jnp.log1p(-u), 1e-10)

    val = s - jnp.log(noise)
    val = jnp.where(col < _NCOLS, val, -jnp.inf)

    cmax = jnp.max(val, axis=1, keepdims=True)
    cand = jnp.where(val == cmax, col, jnp.int32(2**31 - 1))
    cidx = jnp.min(cand, axis=1, keepdims=True)

    @pl.when(j == 0)
    def _():
        bestv_ref[...] = cmax
        idx_ref[...] = cidx

    @pl.when(j > 0)
    def _():
        better = cmax > bestv_ref[...]
        bestv_ref[...] = jnp.where(better, cmax, bestv_ref[...])
        idx_ref[...] = jnp.where(better, cidx, idx_ref[...])


@functools.partial(jax.jit, static_argnames=("interpret",))
def _sample(logits, temperature, interpret=False):
    idx = pl.pallas_call(
        _body,
        grid=(_GRID,),
        in_specs=[
            pl.BlockSpec((_ROWS, _W), lambda j: (0, j)),
            pl.BlockSpec((_ROWS, 1), lambda j: (0, 0)),
        ],
        out_specs=pl.BlockSpec((_ROWS, 1), lambda j: (0, 0)),
        out_shape=jax.ShapeDtypeStruct((_ROWS, 1), jnp.int32),
        scratch_shapes=[pltpu.VMEM((_ROWS, 1), jnp.float32)],
        interpret=interpret,
    )(logits, temperature.reshape(_ROWS, 1))
    return idx[:, 0]


def kernel(logits, temperature):
    return _sample(logits, temperature)


# speedup vs baseline: 1.2483x; 1.2483x over previous
import functools

import jax
import jax.numpy as jnp
from jax.experimental import pallas as pl
from jax.experimental.pallas import tpu as pltpu

_ROWS = 64
_NCOLS = 1_000_000
_W = 8192
_SUB = 512
_NSUB = _W // _SUB
_GRID = (_NCOLS + _W - 1) // _W

_KS1 = 1234
_KS2 = 1234 ^ 0x1BD11BDA
_M32 = 0xFFFFFFFF
_INJ = (
    (_KS1, (_KS2 + 1) & _M32),
    (_KS2, 2),
    (0, _KS1 + 3),
    (_KS1, (_KS2 + 4) & _M32),
    (_KS2, 5),
)
_ROT = ((13, 15, 26, 6), (17, 29, 16, 24))


def _rotl(x, d):
    return (x << d) | (x >> (32 - d))


def _threefry_bits(a):
    x0 = a
    x1 = _rotl(a, 13) ^ a
    for d in (15, 26, 6):
        x0 = x0 + x1
        x1 = _rotl(x1, d) ^ x0
    x0 = x0 + jnp.uint32(_INJ[0][0])
    x1 = x1 + jnp.uint32(_INJ[0][1])
    for g in (1, 2, 3, 4):
        for d in _ROT[g % 2]:
            x0 = x0 + x1
            x1 = _rotl(x1, d) ^ x0
        if _INJ[g][0]:
            x0 = x0 + jnp.uint32(_INJ[g][0])
        x1 = x1 + jnp.uint32(_INJ[g][1])
    return x0 ^ x1


def _body(logits_ref, temp_ref, idx_ref, vmax_ref, vcol_ref):
    j = pl.program_id(0)
    rtemp = 1.0 / temp_ref[...]

    lane = jax.lax.broadcasted_iota(jnp.int32, (_ROWS, _SUB), 1)
    rowoff = jax.lax.broadcasted_iota(jnp.int32, (_ROWS, _SUB), 0) * _NCOLS

    vmax0 = jnp.where(j == 0, jnp.full((_ROWS, _SUB), -jnp.inf, jnp.float32),
                      vmax_ref[...])
    vcol0 = jnp.where(j == 0, jnp.zeros((_ROWS, _SUB), jnp.int32),
                      vcol_ref[...])

    def sub(k, carry):
        vmax, vcol = carry
        c0 = j * _W + k * _SUB
        col = lane + c0
        a = (rowoff + col).astype(jnp.uint32) + jnp.uint32(_KS1)

        bits = _threefry_bits(a)
        fb = (bits >> 9) | jnp.uint32(0x3F800000)
        u = jax.lax.bitcast_convert_type(fb, jnp.float32) - 1.0
        noise = jnp.maximum(-jnp.log1p(-u), 1e-10)

        s = logits_ref[:, pl.ds(k * _SUB, _SUB)] * rtemp
        val = s - jnp.log(noise)
        val = jnp.where(col < _NCOLS, val, -jnp.inf)

        upd = val > vmax
        return jnp.where(upd, val, vmax), jnp.where(upd, col, vcol)

    vmax1, vcol1 = jax.lax.fori_loop(0, _NSUB, sub, (vmax0, vcol0))
    vmax_ref[...] = vmax1
    vcol_ref[...] = vcol1

    @pl.when(j == _GRID - 1)
    def _():
        rmax = jnp.max(vmax1, axis=1, keepdims=True)
        cand = jnp.where(vmax1 == rmax, vcol1, jnp.int32(2**31 - 1))
        idx_ref[...] = jnp.min(cand, axis=1, keepdims=True)


@functools.partial(jax.jit, static_argnames=("interpret",))
def _sample(logits, temperature, interpret=False):
    idx = pl.pallas_call(
        _body,
        grid=(_GRID,),
        in_specs=[
            pl.BlockSpec((_ROWS, _W), lambda j: (0, j)),
            pl.BlockSpec((_ROWS, 1), lambda j: (0, 0)),
        ],
        out_specs=pl.BlockSpec((_ROWS, 1), lambda j: (0, 0)),
        out_shape=jax.ShapeDtypeStruct((_ROWS, 1), jnp.int32),
        scratch_shapes=[
            pltpu.VMEM((_ROWS, _SUB), jnp.float32),
            pltpu.VMEM((_ROWS, _SUB), jnp.int32),
        ],
        interpret=interpret,
    )(logits, temperature.reshape(_ROWS, 1))
    return idx[:, 0]


def kernel(logits, temperature):
    return _sample(logits, temperature)
